# Initial kernel scaffold; baseline (speedup 1.0000x reference)
import jax
import jax.numpy as jnp
from jax.experimental import pallas as pl


def kernel(feat_x, feat_y):
    raise NotImplementedError("write your pallas kernel here")


# speedup vs baseline: 1.5979x; 1.5979x over previous
import jax
import jax.numpy as jnp
from jax import lax
from jax.experimental import pallas as pl
from jax.experimental.pallas import tpu as pltpu

NX = 1024
NY = 100000
C = 64
K = 10
TAU = 0.05
TILE_A = 2048
NY_PAD = 100352
NT_A = NY_PAD // TILE_A
CARRY_W = 128
BIGNEG = -1e30
TILE_B = 2048
NT_B = -(-NY // TILE_B)


def _topk_kernel(x_ref, yt_ref, idx_out_ref, w_out_ref, vals_s, idx_s):
    j = pl.program_id(0)

    @pl.when(j == 0)
    def _init():
        vals_s[...] = jnp.full((NX, CARRY_W), BIGNEG, jnp.float32)
        idx_s[...] = jnp.full((NX, CARRY_W), NY, jnp.int32)

    x = x_ref[...]
    ssx = jnp.sum(x * x, axis=1, keepdims=True)
    xn = x * (1.0 / jnp.maximum(jnp.sqrt(ssx), 1e-12))

    yt = yt_ref[...]
    ssy = jnp.sum(yt * yt, axis=0, keepdims=True)
    ytn = yt * (1.0 / jnp.maximum(jnp.sqrt(ssy), 1e-12))

    sim = jnp.dot(xn, ytn, preferred_element_type=jnp.float32)
    cols = j * TILE_A + lax.broadcasted_iota(jnp.int32, (NX, TILE_A), 1)
    sim = jnp.where(cols < NY, sim, BIGNEG)

    v = jnp.concatenate([vals_s[...], sim], axis=1)
    ii = jnp.concatenate([idx_s[...], cols], axis=1)

    ms = []
    ams = []
    for _ in range(K):
        m = jnp.max(v, axis=1, keepdims=True)
        am = jnp.min(jnp.where(v == m, ii, jnp.int32(2**30)), axis=1,
                     keepdims=True)
        ms.append(m)
        ams.append(am)
        v = jnp.where(ii == am, BIGNEG, v)

    slot = lax.broadcasted_iota(jnp.int32, (NX, CARRY_W), 1)
    newv = jnp.full((NX, CARRY_W), BIGNEG, jnp.float32)
    newi = jnp.full((NX, CARRY_W), NY, jnp.int32)
    for k in range(K):
        newv = jnp.where(slot == k, ms[k], newv)
        newi = jnp.where(slot == k, ams[k], newi)
    vals_s[...] = newv
    idx_s[...] = newi

    @pl.when(j == NT_A - 1)
    def _final():
        m = jnp.max(newv, axis=1, keepdims=True)
        e = jnp.exp((newv - m) / TAU)
        s = jnp.sum(e, axis=1, keepdims=True)
        w_out_ref[...] = e / s
        idx_out_ref[...] = newi


def _scatter_kernel(idx_ref, w_ref, out_ref):
    j = pl.program_id(0)
    cols = j * TILE_B + lax.broadcasted_iota(jnp.int32, (NX, TILE_B), 1)
    idx = idx_ref[...]
    w = w_ref[...]
    acc = jnp.zeros((NX, TILE_B), jnp.float32)
    for k in range(K):
        ik = lax.slice(idx, (0, k), (NX, k + 1))
        wk = lax.slice(w, (0, k), (NX, k + 1))
        acc = acc + jnp.where(cols == ik, wk, 0.0)
    out_ref[...] = acc


def kernel(feat_x, feat_y):
    x = feat_x[0]
    y = feat_y[0]
    yt = jnp.pad(y, ((0, NY_PAD - NY), (0, 0))).T

    idx, w = pl.pallas_call(
        _topk_kernel,
        grid=(NT_A,),
        in_specs=[
            pl.BlockSpec((NX, C), lambda j: (0, 0)),
            pl.BlockSpec((C, TILE_A), lambda j: (0, j)),
        ],
        out_specs=[
            pl.BlockSpec((NX, CARRY_W), lambda j: (0, 0)),
            pl.BlockSpec((NX, CARRY_W), lambda j: (0, 0)),
        ],
        out_shape=[
            jax.ShapeDtypeStruct((NX, CARRY_W), jnp.int32),
            jax.ShapeDtypeStruct((NX, CARRY_W), jnp.float32),
        ],
        scratch_shapes=[
            pltpu.VMEM((NX, CARRY_W), jnp.float32),
            pltpu.VMEM((NX, CARRY_W), jnp.int32),
        ],
        compiler_params=pltpu.CompilerParams(
            dimension_semantics=("arbitrary",)),
    )(x, yt)

    dense = pl.pallas_call(
        _scatter_kernel,
        grid=(NT_B,),
        in_specs=[
            pl.BlockSpec((NX, CARRY_W), lambda j: (0, 0)),
            pl.BlockSpec((NX, CARRY_W), lambda j: (0, 0)),
        ],
        out_specs=pl.BlockSpec((NX, TILE_B), lambda j: (0, j)),
        out_shape=jax.ShapeDtypeStruct((NX, NY), jnp.float32),
        compiler_params=pltpu.CompilerParams(
            dimension_semantics=("arbitrary",)),
    )(idx, w)
    return dense
